# baseline (speedup 1.0000x reference)
import numpy as np
import jax
import jax.numpy as jnp
from jax.experimental import pallas as pl
from jax.experimental.pallas import tpu as pltpu

_N = 4096
_D = 64
_K = 512
_RB = 512
_NB = _N // _RB
_W = 7 * _D


def _init_centroid_ids():
    rng = np.random.default_rng(0)
    return np.asarray(rng.choice(_N, size=_K, replace=False))


def _split3(x):
    hi = x.astype(jnp.bfloat16)
    r1 = x - hi.astype(jnp.float32)
    mid = r1.astype(jnp.bfloat16)
    lo = (r1 - mid.astype(jnp.float32)).astype(jnp.bfloat16)
    return hi, mid, lo


def _kmeans_kernel(it_ref, data_ref, c0_ref, oh_ref, cent_ref, xcat_ref):
    iota_k = jax.lax.broadcasted_iota(jnp.int32, (_RB, _K), 1)

    lane64 = jax.lax.broadcasted_iota(jnp.int32, (_RB, _D), 1)
    ones3 = jnp.where(lane64 < 3, 1.0, 0.0).astype(jnp.bfloat16)

    def stage(b, _):
        x = data_ref[pl.ds(b * _RB, _RB), :]
        xh, xm, xl = _split3(x)
        xcat_ref[pl.ds(b * _RB, _RB), :] = jnp.concatenate(
            [xh, xm, xl, xh, xh, xm, ones3], axis=1)
        return 0

    jax.lax.fori_loop(0, _NB, stage, 0)

    lane64k = jax.lax.broadcasted_iota(jnp.int32, (_K, _D), 1)

    def prep(c):
        ch, cm, cl = _split3(-2.0 * c)
        csq = jnp.sum(c * c, axis=1, keepdims=True)
        qh, qm, ql = _split3(csq)
        csqblk = jnp.where(
            lane64k == 0, qh.astype(jnp.float32),
            jnp.where(lane64k == 1, qm.astype(jnp.float32),
                      jnp.where(lane64k == 2, ql.astype(jnp.float32), 0.0))
        ).astype(jnp.bfloat16)
        return jnp.concatenate([ch, cm, ch, cm, cl, ch, csqblk], axis=1)

    def assign_block(b, ccat):
        xcat = xcat_ref[pl.ds(b * _RB, _RB), :]
        r = jax.lax.dot_general(
            xcat, ccat, (((1,), (1,)), ((), ())),
            preferred_element_type=jnp.float32)
        m = jnp.min(r, axis=1, keepdims=True)
        idx = jnp.min(jnp.where(r == m, iota_k, _K), axis=1, keepdims=True)
        return idx

    def stats_pass(c):
        ccat = prep(c)

        def blk(b2, carry):
            acc, cnt = carry
            for u in range(2):
                b = b2 * 2 + u
                idx = assign_block(b, ccat)
                oh = (iota_k == idx).astype(jnp.bfloat16)
                xupd = xcat_ref[pl.ds(b * _RB, _RB), 0:192]
                acc = acc + jax.lax.dot_general(
                    oh, xupd, (((0,), (0,)), ((), ())),
                    preferred_element_type=jnp.float32)
                cnt = cnt + jnp.sum(oh.astype(jnp.float32), axis=0)
            return acc, cnt

        acc0 = jnp.zeros((_K, 3 * _D), jnp.float32)
        cnt0 = jnp.zeros((_K,), jnp.float32)
        acc, cnt = jax.lax.fori_loop(0, _NB // 2, blk, (acc0, cnt0))
        pseudo = acc[:, 0:_D] + acc[:, _D:2 * _D] + acc[:, 2 * _D:3 * _D]
        return pseudo, cnt

    def iter_body(_, c):
        pseudo, cnt = stats_pass(c)
        return pseudo / cnt[:, None]

    c_final = jax.lax.fori_loop(0, it_ref[0], iter_body, c0_ref[...])
    cent_ref[...] = c_final

    ccat = prep(c_final)

    def final_blk(b, _):
        idx = assign_block(b, ccat)
        oh_ref[pl.ds(b * _RB, _RB), :] = (iota_k == idx).astype(jnp.float32)
        return 0

    jax.lax.fori_loop(0, _NB, final_blk, 0)


def kernel(data, iteration):
    c0 = jnp.take(data, jnp.asarray(_init_centroid_ids()), axis=0)
    it = jnp.asarray(iteration, jnp.int32).reshape(1)
    onehot, centroids = pl.pallas_call(
        _kmeans_kernel,
        in_specs=[
            pl.BlockSpec(memory_space=pltpu.SMEM),
            pl.BlockSpec(memory_space=pltpu.VMEM),
            pl.BlockSpec(memory_space=pltpu.VMEM),
        ],
        out_specs=[
            pl.BlockSpec(memory_space=pltpu.VMEM),
            pl.BlockSpec(memory_space=pltpu.VMEM),
        ],
        out_shape=[
            jax.ShapeDtypeStruct((_N, _K), jnp.float32),
            jax.ShapeDtypeStruct((_K, _D), jnp.float32),
        ],
        scratch_shapes=[pltpu.VMEM((_N, _W), jnp.bfloat16)],
    )(it, data, c0)
    return onehot, centroids


# speedup vs baseline: 1.2175x; 1.2175x over previous
import numpy as np
import jax
import jax.numpy as jnp
from jax.experimental import pallas as pl
from jax.experimental.pallas import tpu as pltpu

_N = 4096
_D = 64
_K = 512
_RB = 512
_NB = _N // _RB
_W = 7 * _D


def _init_centroid_ids():
    rng = np.random.default_rng(0)
    return np.asarray(rng.choice(_N, size=_K, replace=False))


def _split3(x):
    hi = x.astype(jnp.bfloat16)
    r1 = x - hi.astype(jnp.float32)
    mid = r1.astype(jnp.bfloat16)
    lo = (r1 - mid.astype(jnp.float32)).astype(jnp.bfloat16)
    return hi, mid, lo


def _kmeans_kernel(it_ref, data_ref, c0_ref, oh_ref, cent_ref, xcat_ref):
    iota_k = jax.lax.broadcasted_iota(jnp.int32, (_RB, _K), 1)

    lane64 = jax.lax.broadcasted_iota(jnp.int32, (_RB, _D), 1)
    ones3 = jnp.where(lane64 < 3, 1.0, 0.0).astype(jnp.bfloat16)

    def stage(b, _):
        x = data_ref[pl.ds(b * _RB, _RB), :]
        xh, xm, xl = _split3(x)
        xcat_ref[pl.ds(b * _RB, _RB), :] = jnp.concatenate(
            [xh, xm, xl, xh, xh, xm, ones3], axis=1)
        return 0

    jax.lax.fori_loop(0, _NB, stage, 0)

    lane64k = jax.lax.broadcasted_iota(jnp.int32, (_K, _D), 1)

    def prep(c):
        ch, cm, cl = _split3(-2.0 * c)
        csq = jnp.sum(c * c, axis=1, keepdims=True)
        qh, qm, ql = _split3(csq)
        csqblk = jnp.where(
            lane64k == 0, qh.astype(jnp.float32),
            jnp.where(lane64k == 1, qm.astype(jnp.float32),
                      jnp.where(lane64k == 2, ql.astype(jnp.float32), 0.0))
        ).astype(jnp.bfloat16)
        return jnp.concatenate([ch, cm, ch, cm, cl, ch, csqblk], axis=1)

    def assign_block(b, ccat):
        xcat = xcat_ref[pl.ds(b * _RB, _RB), :]
        r = jax.lax.dot_general(
            xcat, ccat, (((1,), (1,)), ((), ())),
            preferred_element_type=jnp.float32)
        m = jnp.min(r, axis=1, keepdims=True)
        idx = jnp.min(jnp.where(r == m, iota_k, _K), axis=1, keepdims=True)
        return idx

    def stats_pass(c):
        ccat = prep(c)

        def blk(b2, carry):
            acc, cnt = carry
            for u in range(4):
                b = b2 * 4 + u
                idx = assign_block(b, ccat)
                oh = (iota_k == idx).astype(jnp.bfloat16)
                xupd = xcat_ref[pl.ds(b * _RB, _RB), 0:192]
                acc = acc + jax.lax.dot_general(
                    oh, xupd, (((0,), (0,)), ((), ())),
                    preferred_element_type=jnp.float32)
                cnt = cnt + jnp.sum(oh.astype(jnp.float32), axis=0)
            return acc, cnt

        acc0 = jnp.zeros((_K, 3 * _D), jnp.float32)
        cnt0 = jnp.zeros((_K,), jnp.float32)
        acc, cnt = jax.lax.fori_loop(0, _NB // 4, blk, (acc0, cnt0))
        pseudo = acc[:, 0:_D] + acc[:, _D:2 * _D] + acc[:, 2 * _D:3 * _D]
        return pseudo, cnt

    def iter_body(_, c):
        pseudo, cnt = stats_pass(c)
        return pseudo / cnt[:, None]

    c_final = jax.lax.fori_loop(0, it_ref[0], iter_body, c0_ref[...])
    cent_ref[...] = c_final

    ccat = prep(c_final)

    def final_blk(b, _):
        idx = assign_block(b, ccat)
        oh_ref[pl.ds(b * _RB, _RB), :] = (iota_k == idx).astype(jnp.float32)
        return 0

    jax.lax.fori_loop(0, _NB, final_blk, 0)


def kernel(data, iteration):
    c0 = jnp.take(data, jnp.asarray(_init_centroid_ids()), axis=0)
    it = jnp.asarray(iteration, jnp.int32).reshape(1)
    onehot, centroids = pl.pallas_call(
        _kmeans_kernel,
        in_specs=[
            pl.BlockSpec(memory_space=pltpu.SMEM),
            pl.BlockSpec(memory_space=pltpu.VMEM),
            pl.BlockSpec(memory_space=pltpu.VMEM),
        ],
        out_specs=[
            pl.BlockSpec(memory_space=pltpu.VMEM),
            pl.BlockSpec(memory_space=pltpu.VMEM),
        ],
        out_shape=[
            jax.ShapeDtypeStruct((_N, _K), jnp.float32),
            jax.ShapeDtypeStruct((_K, _D), jnp.float32),
        ],
        scratch_shapes=[pltpu.VMEM((_N, _W), jnp.bfloat16)],
    )(it, data, c0)
    return onehot, centroids


# speedup vs baseline: 1.4746x; 1.2111x over previous
import numpy as np
import jax
import jax.numpy as jnp
from jax.experimental import pallas as pl
from jax.experimental.pallas import tpu as pltpu

_N = 4096
_D = 64
_K = 512
_RB = 512
_NB = _N // _RB
_W = 7 * _D


def _init_centroid_ids():
    rng = np.random.default_rng(0)
    return np.asarray(rng.choice(_N, size=_K, replace=False))


def _split3(x):
    hi = x.astype(jnp.bfloat16)
    r1 = x - hi.astype(jnp.float32)
    mid = r1.astype(jnp.bfloat16)
    lo = (r1 - mid.astype(jnp.float32)).astype(jnp.bfloat16)
    return hi, mid, lo


def _kmeans_kernel(it_ref, data_ref, c0_ref, oh_ref, cent_ref, xcat_ref):
    iota_k = jax.lax.broadcasted_iota(jnp.int32, (_RB, _K), 1)

    lane64 = jax.lax.broadcasted_iota(jnp.int32, (_RB, _D), 1)
    ones3 = jnp.where(lane64 < 3, 1.0, 0.0).astype(jnp.bfloat16)

    def stage(b, _):
        x = data_ref[pl.ds(b * _RB, _RB), :]
        xh, xm, xl = _split3(x)
        xcat_ref[pl.ds(b * _RB, _RB), :] = jnp.concatenate(
            [xh, xm, xl, xh, xh, xm, ones3], axis=1)
        return 0

    jax.lax.fori_loop(0, _NB, stage, 0)

    lane64k = jax.lax.broadcasted_iota(jnp.int32, (_K, _D), 1)

    def prep(c):
        ch, cm, cl = _split3(-2.0 * c)
        csq = jnp.sum(c * c, axis=1, keepdims=True)
        qh, qm, ql = _split3(csq)
        csqblk = jnp.where(
            lane64k == 0, qh.astype(jnp.float32),
            jnp.where(lane64k == 1, qm.astype(jnp.float32),
                      jnp.where(lane64k == 2, ql.astype(jnp.float32), 0.0))
        ).astype(jnp.bfloat16)
        return jnp.concatenate([ch, cm, ch, cm, cl, ch, csqblk], axis=1)

    def assign_block(b, ccat):
        xcat = xcat_ref[pl.ds(b * _RB, _RB), :]
        r = jax.lax.dot_general(
            xcat, ccat, (((1,), (1,)), ((), ())),
            preferred_element_type=jnp.float32)
        m = jnp.min(r, axis=1, keepdims=True)
        idx = jnp.min(jnp.where(r == m, iota_k, _K), axis=1, keepdims=True)
        return idx

    def stats_pass(c):
        ccat = prep(c)

        def blk(b2, carry):
            acc, cnt = carry
            for u in range(8):
                b = b2 * 8 + u
                idx = assign_block(b, ccat)
                oh = (iota_k == idx).astype(jnp.bfloat16)
                xupd = xcat_ref[pl.ds(b * _RB, _RB), 0:192]
                acc = acc + jax.lax.dot_general(
                    oh, xupd, (((0,), (0,)), ((), ())),
                    preferred_element_type=jnp.float32)
                cnt = cnt + jnp.sum(oh.astype(jnp.float32), axis=0)
            return acc, cnt

        acc0 = jnp.zeros((_K, 3 * _D), jnp.float32)
        cnt0 = jnp.zeros((_K,), jnp.float32)
        acc, cnt = jax.lax.fori_loop(0, _NB // 8, blk, (acc0, cnt0))
        pseudo = acc[:, 0:_D] + acc[:, _D:2 * _D] + acc[:, 2 * _D:3 * _D]
        return pseudo, cnt

    def iter_body(_, c):
        pseudo, cnt = stats_pass(c)
        return pseudo / cnt[:, None]

    c_final = jax.lax.fori_loop(0, it_ref[0], iter_body, c0_ref[...])
    cent_ref[...] = c_final

    ccat = prep(c_final)

    def final_blk(b, _):
        idx = assign_block(b, ccat)
        oh_ref[pl.ds(b * _RB, _RB), :] = (iota_k == idx).astype(jnp.float32)
        return 0

    jax.lax.fori_loop(0, _NB, final_blk, 0)


def kernel(data, iteration):
    c0 = jnp.take(data, jnp.asarray(_init_centroid_ids()), axis=0)
    it = jnp.asarray(iteration, jnp.int32).reshape(1)
    onehot, centroids = pl.pallas_call(
        _kmeans_kernel,
        in_specs=[
            pl.BlockSpec(memory_space=pltpu.SMEM),
            pl.BlockSpec(memory_space=pltpu.VMEM),
            pl.BlockSpec(memory_space=pltpu.VMEM),
        ],
        out_specs=[
            pl.BlockSpec(memory_space=pltpu.VMEM),
            pl.BlockSpec(memory_space=pltpu.VMEM),
        ],
        out_shape=[
            jax.ShapeDtypeStruct((_N, _K), jnp.float32),
            jax.ShapeDtypeStruct((_K, _D), jnp.float32),
        ],
        scratch_shapes=[pltpu.VMEM((_N, _W), jnp.bfloat16)],
    )(it, data, c0)
    return onehot, centroids
